# Initial kernel scaffold; baseline (speedup 1.0000x reference)
import jax
import jax.numpy as jnp
from jax.experimental import pallas as pl


def kernel(x, pos, batch, gW0, gb0, gW1, gb1, gW2, gb2, gW3, gb3, gWo, gbo, fW0, fb0, fW1, fb1, fW2, fb2, fW3, fb3, fWo, fbo):
    raise NotImplementedError("write your pallas kernel here")


# speedup vs baseline: 3.0426x; 3.0426x over previous
import functools

import jax
import jax.numpy as jnp
from jax import lax
from jax.experimental import pallas as pl
from jax.experimental.pallas import tpu as pltpu

N = 10000
F = 256
B = 256
K = 3
NB = 256
NBLK = 40
NPAD = NBLK * NB
NEG = -3.0e38
BIGI = 2 ** 30


def _leaky(h):
    return jnp.where(h >= 0, h, jnp.float32(0.01) * h)


def _phase1_body(x_ref, br_ref, bnr_ref, bc_ref,
                 gW0_ref, gb0_ref, gW1_ref, gb1_ref, gW2_ref, gb2_ref,
                 gW3_ref, gb3_ref, gWo_ref, gbo_ref,
                 mx_o, mean_o, sm_o, attn_o, topi_o, cnt_o,
                 sm_s, mx_s, A_s, s_s, m_s, cnt_s, topv_s, topi_s,
                 carry_s, carryb_s):
    i = pl.program_id(0)

    @pl.when(i == 0)
    def _init():
        sm_s[...] = jnp.zeros_like(sm_s)
        mx_s[...] = jnp.zeros_like(mx_s)
        A_s[...] = jnp.zeros_like(A_s)
        s_s[...] = jnp.zeros_like(s_s)
        cnt_s[...] = jnp.zeros_like(cnt_s)
        m_s[...] = jnp.full_like(m_s, NEG)
        topv_s[...] = jnp.full_like(topv_s, NEG)
        topi_s[...] = -1 - lax.broadcasted_iota(jnp.int32, topi_s.shape, 1)
        carry_s[...] = jnp.full_like(carry_s, NEG)
        carryb_s[...] = jnp.full_like(carryb_s, -1)

    xb = x_ref[...]
    bb = br_ref[0]
    bn = bnr_ref[0]
    bc = bc_ref[0]

    segs = lax.broadcasted_iota(jnp.int32, (B, NB), 0)
    Mb = segs == bb
    Mf = Mb.astype(jnp.float32)

    cnt_s[...] = cnt_s[...] + jnp.sum(Mf, axis=1, keepdims=True)
    sm_s[...] = sm_s[...] + jnp.dot(Mf, xb, preferred_element_type=jnp.float32, precision=lax.Precision.HIGHEST)

    same0 = carryb_s[...] == bc[0:1, :]
    row0 = jnp.where(same0, jnp.maximum(xb[0:1, :], carry_s[...]), xb[0:1, :])
    sc = jnp.concatenate([row0, xb[1:, :]], axis=0)
    d = 1
    while d < NB:
        shifted = jnp.concatenate(
            [jnp.full((d, F), NEG, jnp.float32), sc[:NB - d, :]], axis=0)
        bshift = jnp.concatenate(
            [jnp.full((d, 1), -7, jnp.int32), bc[:NB - d, :]], axis=0)
        ok = bc == bshift
        sc = jnp.where(ok, jnp.maximum(sc, shifted), sc)
        d *= 2
    carry_s[...] = sc[NB - 1:NB, :]
    carryb_s[...] = bc[NB - 1:NB, :]
    is_end = (bb != bn).astype(jnp.float32)
    S = Mf * is_end
    mx_s[...] = mx_s[...] + jnp.dot(S, sc, preferred_element_type=jnp.float32, precision=lax.Precision.HIGHEST)

    h = xb
    for W_ref, b_ref in ((gW0_ref, gb0_ref), (gW1_ref, gb1_ref),
                         (gW2_ref, gb2_ref), (gW3_ref, gb3_ref)):
        h = _leaky(jnp.dot(h, W_ref[...], preferred_element_type=jnp.float32, precision=lax.Precision.HIGHEST)
                   + b_ref[...])
    g_row = lax.dot_general(gWo_ref[...], h, (((0,), (1,)), ((), ())),
                            preferred_element_type=jnp.float32, precision=lax.Precision.HIGHEST) + gbo_ref[...]

    GM = jnp.where(Mb, g_row, NEG)
    blkmax = jnp.max(GM, axis=1, keepdims=True)
    m_new = jnp.maximum(m_s[...], blkmax)
    scale = jnp.where(m_s[...] <= NEG, jnp.float32(0.0),
                      jnp.exp(m_s[...] - m_new))
    Wm = jnp.where(Mb, jnp.exp(g_row - m_new), jnp.float32(0.0))
    s_s[...] = s_s[...] * scale + jnp.sum(Wm, axis=1, keepdims=True)
    A_s[...] = A_s[...] * scale + jnp.dot(Wm, xb,
                                          preferred_element_type=jnp.float32, precision=lax.Precision.HIGHEST)
    m_s[...] = m_new

    sel = (lax.broadcasted_iota(jnp.int32, (1, F), 1) == F - 1
           ).astype(jnp.float32)
    krow = lax.dot_general(sel, xb, (((1,), (1,)), ((), ())),
                           preferred_element_type=jnp.float32, precision=lax.Precision.HIGHEST)
    KM = jnp.where(Mb, krow, NEG)
    bidx = lax.broadcasted_iota(jnp.int32, (B, NB), 1) + i * NB
    cand_v, cand_i = [], []
    for _ in range(K):
        v = jnp.max(KM, axis=1, keepdims=True)
        ii = jnp.min(jnp.where(KM == v, bidx, BIGI), axis=1, keepdims=True)
        cand_v.append(v)
        cand_i.append(ii)
        KM = jnp.where(bidx == ii, NEG, KM)
    vals6 = jnp.concatenate([topv_s[:, :K]] + cand_v, axis=1)
    idxs6 = jnp.concatenate([topi_s[:, :K]] + cand_i, axis=1)
    new_v, new_i = [], []
    for _ in range(K):
        v = jnp.max(vals6, axis=1, keepdims=True)
        ii = jnp.min(jnp.where(vals6 == v, idxs6, BIGI), axis=1, keepdims=True)
        new_v.append(v)
        new_i.append(ii)
        vals6 = jnp.where(idxs6 == ii, NEG, vals6)
    pad_v = jnp.full((B, 1), NEG, jnp.float32)
    pad_i = jnp.full((B, 1), -9, jnp.int32)
    topv_s[...] = jnp.concatenate(new_v + [pad_v], axis=1)
    topi_s[...] = jnp.concatenate(new_i + [pad_i], axis=1)

    @pl.when(i == NBLK - 1)
    def _fin():
        cnt = cnt_s[...]
        sm = sm_s[...]
        mx_o[...] = mx_s[...]
        sm_o[...] = sm
        mean_o[...] = sm / jnp.maximum(cnt, 1.0)
        attn_o[...] = A_s[...] / (s_s[...] + jnp.float32(1e-16))
        cnt_o[...] = cnt
        topi_o[...] = topi_s[...]


def _phase2_body(x_ref, topi_ref, cnt_ref,
                 mx_ref, mean_ref, sm_ref, attn_ref,
                 fW0_ref, fb0_ref, fW1_ref, fb1_ref, fW2_ref, fb2_ref,
                 fW3_ref, fb3_ref, fWo_ref, fbo_ref,
                 out_o, sf0_s, sf1_s, sf2_s):
    i = pl.program_id(0)

    @pl.when(i == 0)
    def _init():
        sf0_s[...] = jnp.zeros_like(sf0_s)
        sf1_s[...] = jnp.zeros_like(sf1_s)
        sf2_s[...] = jnp.zeros_like(sf2_s)

    @pl.when(i < NBLK)
    def _acc():
        xb = x_ref[...]
        bidx = lax.broadcasted_iota(jnp.int32, (B, NB), 1) + i * NB
        cnt = cnt_ref[...]
        for k, sf in enumerate((sf0_s, sf1_s, sf2_s)):
            tk = topi_ref[:, k:k + 1]
            Sk = ((bidx == tk) & (cnt > jnp.float32(k))).astype(jnp.float32)
            sf[...] = sf[...] + jnp.dot(Sk, xb,
                                        preferred_element_type=jnp.float32, precision=lax.Precision.HIGHEST)

    @pl.when(i == NBLK)
    def _mlp():
        h = (jnp.dot(mx_ref[...], fW0_ref[0], preferred_element_type=jnp.float32, precision=lax.Precision.HIGHEST)
             + jnp.dot(mean_ref[...], fW0_ref[1], preferred_element_type=jnp.float32, precision=lax.Precision.HIGHEST)
             + jnp.dot(sm_ref[...], fW0_ref[2], preferred_element_type=jnp.float32, precision=lax.Precision.HIGHEST)
             + jnp.dot(attn_ref[...], fW0_ref[3], preferred_element_type=jnp.float32, precision=lax.Precision.HIGHEST)
             + jnp.dot(sf0_s[...], fW0_ref[4], preferred_element_type=jnp.float32, precision=lax.Precision.HIGHEST)
             + jnp.dot(sf1_s[...], fW0_ref[5], preferred_element_type=jnp.float32, precision=lax.Precision.HIGHEST)
             + jnp.dot(sf2_s[...], fW0_ref[6], preferred_element_type=jnp.float32, precision=lax.Precision.HIGHEST)
             + fb0_ref[...])
        h = _leaky(h)
        for W_ref, b_ref in ((fW1_ref, fb1_ref), (fW2_ref, fb2_ref),
                             (fW3_ref, fb3_ref)):
            h = _leaky(jnp.dot(h, W_ref[...],
                               preferred_element_type=jnp.float32, precision=lax.Precision.HIGHEST) + b_ref[...])
        out_o[...] = jnp.dot(h, fWo_ref[...],
                             preferred_element_type=jnp.float32, precision=lax.Precision.HIGHEST) + fbo_ref[...]


def _const2(shape):
    return pl.BlockSpec(shape, lambda i: (0,) * len(shape))


def kernel(x, pos, batch, gW0, gb0, gW1, gb1, gW2, gb2, gW3, gb3, gWo, gbo,
           fW0, fb0, fW1, fb1, fW2, fb2, fW3, fb3, fWo, fbo):
    del pos
    xp = jnp.pad(x, ((0, NPAD - N), (0, 0)))
    bp = jnp.pad(batch.astype(jnp.int32), (0, NPAD - N), constant_values=B)
    bnx = jnp.concatenate([bp[1:], jnp.full((1,), B + 1, jnp.int32)])
    br = bp.reshape(NBLK, 1, NB)
    bnr = bnx.reshape(NBLK, 1, NB)
    bc = bp.reshape(NBLK, NB, 1)

    f32 = jnp.float32
    phase1 = pl.pallas_call(
        _phase1_body,
        grid=(NBLK,),
        in_specs=[
            pl.BlockSpec((NB, F), lambda i: (i, 0)),
            pl.BlockSpec((1, 1, NB), lambda i: (i, 0, 0)),
            pl.BlockSpec((1, 1, NB), lambda i: (i, 0, 0)),
            pl.BlockSpec((1, NB, 1), lambda i: (i, 0, 0)),
            _const2((F, F)), _const2((1, F)),
            _const2((F, F)), _const2((1, F)),
            _const2((F, F)), _const2((1, F)),
            _const2((F, F)), _const2((1, F)),
            _const2((F, 1)), _const2((1, 1)),
        ],
        out_specs=[
            _const2((B, F)), _const2((B, F)), _const2((B, F)),
            _const2((B, F)), _const2((B, K + 1)), _const2((B, 1)),
        ],
        out_shape=[
            jax.ShapeDtypeStruct((B, F), f32),
            jax.ShapeDtypeStruct((B, F), f32),
            jax.ShapeDtypeStruct((B, F), f32),
            jax.ShapeDtypeStruct((B, F), f32),
            jax.ShapeDtypeStruct((B, K + 1), jnp.int32),
            jax.ShapeDtypeStruct((B, 1), f32),
        ],
        scratch_shapes=[
            pltpu.VMEM((B, F), f32), pltpu.VMEM((B, F), f32),
            pltpu.VMEM((B, F), f32), pltpu.VMEM((B, 1), f32),
            pltpu.VMEM((B, 1), f32), pltpu.VMEM((B, 1), f32),
            pltpu.VMEM((B, K + 1), f32), pltpu.VMEM((B, K + 1), jnp.int32),
            pltpu.VMEM((1, F), f32), pltpu.VMEM((1, 1), jnp.int32),
        ],
    )
    mx, mean, sm, attn, topi, cnt = phase1(
        xp, br, bnr, bc,
        gW0, gb0.reshape(1, F), gW1, gb1.reshape(1, F),
        gW2, gb2.reshape(1, F), gW3, gb3.reshape(1, F),
        gWo, gbo.reshape(1, 1))

    phase2 = pl.pallas_call(
        _phase2_body,
        grid=(NBLK + 1,),
        in_specs=[
            pl.BlockSpec((NB, F), lambda i: (jnp.minimum(i, NBLK - 1), 0)),
            _const2((B, K + 1)), _const2((B, 1)),
            _const2((B, F)), _const2((B, F)), _const2((B, F)), _const2((B, F)),
            _const2((7, F, F)), _const2((1, F)),
            _const2((F, F)), _const2((1, F)),
            _const2((F, F)), _const2((1, F)),
            _const2((F, F)), _const2((1, F)),
            _const2((F, F)), _const2((1, F)),
        ],
        out_specs=_const2((B, F)),
        out_shape=jax.ShapeDtypeStruct((B, F), f32),
        scratch_shapes=[
            pltpu.VMEM((B, F), f32), pltpu.VMEM((B, F), f32),
            pltpu.VMEM((B, F), f32),
        ],
    )
    out = phase2(
        xp, topi, cnt, mx, mean, sm, attn,
        fW0.reshape(7, F, F), fb0.reshape(1, F),
        fW1, fb1.reshape(1, F), fW2, fb2.reshape(1, F),
        fW3, fb3.reshape(1, F), fWo, fbo.reshape(1, F))
    return out


# speedup vs baseline: 6.2308x; 2.0478x over previous
import jax
import jax.numpy as jnp
from jax import lax
from jax.experimental import pallas as pl
from jax.experimental.pallas import tpu as pltpu

N = 10000
F = 256
B = 256
K = 3
NB = 256
NBLK = 40
NPAD = NBLK * NB
NEG = -3.0e38
BIGI = 2 ** 30

_D0 = (((0,), (0,)), ((), ()))


def _leaky(h):
    return jnp.where(h >= 0, h, jnp.float32(0.01) * h)


def _split(a):
    hi = a.astype(jnp.bfloat16)
    lo = (a - hi.astype(jnp.float32)).astype(jnp.bfloat16)
    return hi, lo


def _dg(a, b, dims):
    return lax.dot_general(a, b, dims, preferred_element_type=jnp.float32)


def _mm3(ah, al, bh, bl):
    d = lambda p, q: jnp.dot(p, q, preferred_element_type=jnp.float32)
    return d(ah, bh) + d(ah, bl) + d(al, bh)


def _mm2(ah, al, sel_b):
    return _dg(ah, sel_b, _D0) + _dg(al, sel_b, _D0)


def _phase1_body(x_ref, bc_ref, bnc_ref,
                 gW0h_ref, gW0l_ref, gb0_ref, gW1h_ref, gW1l_ref, gb1_ref,
                 gW2h_ref, gW2l_ref, gb2_ref, gW3h_ref, gW3l_ref, gb3_ref,
                 gwo_ref, gbo_ref,
                 mx_o, mean_o, sm_o, attn_o, topi_o,
                 sm_s, mx_s, At_s, s_s, m_s, cnt_s, topv_s, topi_s,
                 carry_s, carryb_s):
    i = pl.program_id(0)

    @pl.when(i == 0)
    def _init():
        sm_s[...] = jnp.zeros_like(sm_s)
        mx_s[...] = jnp.zeros_like(mx_s)
        At_s[...] = jnp.zeros_like(At_s)
        s_s[...] = jnp.zeros_like(s_s)
        cnt_s[...] = jnp.zeros_like(cnt_s)
        m_s[...] = jnp.full_like(m_s, NEG)
        topv_s[...] = jnp.full_like(topv_s, NEG)
        topi_s[...] = -1 - lax.broadcasted_iota(jnp.int32, topi_s.shape, 0)
        carry_s[...] = jnp.full_like(carry_s, NEG)
        carryb_s[...] = jnp.full_like(carryb_s, -1)

    xb = x_ref[...]
    bc = bc_ref[0]
    bnc = bnc_ref[0]
    xh, xl = _split(xb)

    MT = lax.broadcasted_iota(jnp.int32, (NB, B), 1) == bc
    MTb = MT.astype(jnp.bfloat16)

    cnt_s[...] = cnt_s[...] + jnp.sum(MT.astype(jnp.float32), axis=0,
                                      keepdims=True)
    sm_s[...] = sm_s[...] + _dg(MTb, xh, _D0) + _dg(MTb, xl, _D0)

    same0 = carryb_s[...] == bc[0:1, :]
    row0 = jnp.where(same0, jnp.maximum(xb[0:1, :], carry_s[...]), xb[0:1, :])
    sc = jnp.concatenate([row0, xb[1:, :]], axis=0)
    d = 1
    while d < NB:
        shifted = jnp.concatenate(
            [jnp.full((d, F), NEG, jnp.float32), sc[:NB - d, :]], axis=0)
        bshift = jnp.concatenate(
            [jnp.full((d, 1), -7, jnp.int32), bc[:NB - d, :]], axis=0)
        ok = bc == bshift
        sc = jnp.where(ok, jnp.maximum(sc, shifted), sc)
        d *= 2
    carry_s[...] = sc[NB - 1:NB, :]
    carryb_s[...] = bc[NB - 1:NB, :]
    sch, scl = _split(sc)
    St = MTb * (bc != bnc).astype(jnp.bfloat16)
    mx_s[...] = mx_s[...] + _dg(St, sch, _D0) + _dg(St, scl, _D0)

    h = xb
    for Wh_ref, Wl_ref, b_ref in (
            (gW0h_ref, gW0l_ref, gb0_ref), (gW1h_ref, gW1l_ref, gb1_ref),
            (gW2h_ref, gW2l_ref, gb2_ref), (gW3h_ref, gW3l_ref, gb3_ref)):
        hh, hl = _split(h)
        t = (jnp.dot(hh, Wh_ref[...], preferred_element_type=jnp.float32)
             + jnp.dot(hh, Wl_ref[...], preferred_element_type=jnp.float32)
             + jnp.dot(hl, Wh_ref[...], preferred_element_type=jnp.float32))
        h = _leaky(t + b_ref[...])
    g = jnp.sum(h * gwo_ref[...], axis=1, keepdims=True) + gbo_ref[...]

    GM = jnp.where(MT, g, NEG)
    blkmax = jnp.max(GM, axis=0, keepdims=True)
    m_new = jnp.maximum(m_s[...], blkmax)
    scale = jnp.where(m_s[...] <= NEG, jnp.float32(0.0),
                      jnp.exp(m_s[...] - m_new))
    Wt = jnp.where(MT, jnp.exp(g - m_new), jnp.float32(0.0))
    s_s[...] = s_s[...] * scale + jnp.sum(Wt, axis=0, keepdims=True)
    Wth, Wtl = _split(Wt)
    At_s[...] = (At_s[...] * scale
                 + _dg(xh, Wth, _D0) + _dg(xh, Wtl, _D0) + _dg(xl, Wth, _D0))
    m_s[...] = m_new

    key = xb[:, F - 1:F]
    KM = jnp.where(MT, key, NEG)
    bidx = lax.broadcasted_iota(jnp.int32, (NB, B), 0) + i * NB
    cand_v, cand_i = [], []
    for _ in range(K):
        v = jnp.max(KM, axis=0, keepdims=True)
        ii = jnp.min(jnp.where(KM == v, bidx, BIGI), axis=0, keepdims=True)
        cand_v.append(v)
        cand_i.append(ii)
        KM = jnp.where(bidx == ii, NEG, KM)
    vals6 = jnp.concatenate([topv_s[:K]] + cand_v, axis=0)
    idxs6 = jnp.concatenate([topi_s[:K]] + cand_i, axis=0)
    new_v, new_i = [], []
    for _ in range(K):
        v = jnp.max(vals6, axis=0, keepdims=True)
        ii = jnp.min(jnp.where(vals6 == v, idxs6, BIGI), axis=0, keepdims=True)
        new_v.append(v)
        new_i.append(ii)
        vals6 = jnp.where(idxs6 == ii, NEG, vals6)
    pad_v = jnp.full((1, B), NEG, jnp.float32)
    pad_i = jnp.full((1, B), -9, jnp.int32)
    topv_s[...] = jnp.concatenate(new_v + [pad_v], axis=0)
    topi_s[...] = jnp.concatenate(new_i + [pad_i], axis=0)

    @pl.when(i == NBLK - 1)
    def _fin():
        cnt = cnt_s[...]
        Ieq = (lax.broadcasted_iota(jnp.int32, (B, B), 0)
               == lax.broadcasted_iota(jnp.int32, (B, B), 1))
        Ib = Ieq.astype(jnp.bfloat16)
        cntc = jnp.sum(jnp.where(Ieq, jnp.broadcast_to(cnt, (B, B)), 0.0),
                       axis=1, keepdims=True)
        sm = sm_s[...]
        mx_o[...] = mx_s[...]
        sm_o[...] = sm
        mean_o[...] = sm / jnp.maximum(cntc, 1.0)
        attn_t = At_s[...] / (s_s[...] + jnp.float32(1e-16))
        ath, atl = _split(attn_t)
        attn_o[...] = _mm2(ath, atl, Ib)
        kio = lax.broadcasted_iota(jnp.int32, (K + 1, B), 0).astype(jnp.float32)
        topi_o[...] = jnp.where(kio < cnt, topi_s[...], N)


def _phase2_body(xf_ref, topi_ref, mx_ref, mean_ref, sm_ref, attn_ref,
                 fW0h_ref, fW0l_ref, fb0_ref, fW1h_ref, fW1l_ref, fb1_ref,
                 fW2h_ref, fW2l_ref, fb2_ref, fW3h_ref, fW3l_ref, fb3_ref,
                 fWoh_ref, fWol_ref, fbo_ref,
                 out_o, sf0_s, sf1_s, sf2_s):
    sfs = (sf0_s, sf1_s, sf2_s)

    def gather_body(b, _):
        for k in range(K):
            idx = topi_ref[k, b]
            sfs[k][pl.ds(b, 1), :] = xf_ref[pl.ds(idx, 1), :]
        return 0

    lax.fori_loop(0, B, gather_body, 0)

    parts = (mx_ref[...], mean_ref[...], sm_ref[...], attn_ref[...],
             sf0_s[...], sf1_s[...], sf2_s[...])
    acc = fb0_ref[...]
    for j, p in enumerate(parts):
        ph, pll = _split(p)
        acc = acc + _mm3(ph, pll, fW0h_ref[j], fW0l_ref[j])
    h = _leaky(acc)
    for Wh_ref, Wl_ref, b_ref in (
            (fW1h_ref, fW1l_ref, fb1_ref), (fW2h_ref, fW2l_ref, fb2_ref),
            (fW3h_ref, fW3l_ref, fb3_ref)):
        hh, hl = _split(h)
        h = _leaky(_mm3(hh, hl, Wh_ref[...], Wl_ref[...]) + b_ref[...])
    hh, hl = _split(h)
    out_o[...] = _mm3(hh, hl, fWoh_ref[...], fWol_ref[...]) + fbo_ref[...]


def _const(shape):
    return pl.BlockSpec(shape, lambda i: (0,) * len(shape))


def _wsplit(W):
    hi = W.astype(jnp.bfloat16)
    lo = (W - hi.astype(jnp.float32)).astype(jnp.bfloat16)
    return hi, lo


def kernel(x, pos, batch, gW0, gb0, gW1, gb1, gW2, gb2, gW3, gb3, gWo, gbo,
           fW0, fb0, fW1, fb1, fW2, fb2, fW3, fb3, fWo, fbo):
    del pos
    f32 = jnp.float32
    xp = jnp.pad(x, ((0, NPAD - N), (0, 0)))
    bp = jnp.pad(batch.astype(jnp.int32), (0, NPAD - N), constant_values=B)
    bnx = jnp.concatenate([bp[1:], jnp.full((1,), B + 1, jnp.int32)])
    bc = bp.reshape(NBLK, NB, 1)
    bnc = bnx.reshape(NBLK, NB, 1)

    gs = [_wsplit(W) for W in (gW0, gW1, gW2, gW3)]

    phase1 = pl.pallas_call(
        _phase1_body,
        grid=(NBLK,),
        in_specs=[
            pl.BlockSpec((NB, F), lambda i: (i, 0)),
            pl.BlockSpec((1, NB, 1), lambda i: (i, 0, 0)),
            pl.BlockSpec((1, NB, 1), lambda i: (i, 0, 0)),
        ] + [_const((F, F)), _const((F, F)), _const((1, F))] * 4
          + [_const((1, F)), _const((1, 1))],
        out_specs=[
            _const((B, F)), _const((B, F)), _const((B, F)), _const((B, F)),
            _const((K + 1, B)),
        ],
        out_shape=[
            jax.ShapeDtypeStruct((B, F), f32),
            jax.ShapeDtypeStruct((B, F), f32),
            jax.ShapeDtypeStruct((B, F), f32),
            jax.ShapeDtypeStruct((B, F), f32),
            jax.ShapeDtypeStruct((K + 1, B), jnp.int32),
        ],
        scratch_shapes=[
            pltpu.VMEM((B, F), f32), pltpu.VMEM((B, F), f32),
            pltpu.VMEM((F, B), f32), pltpu.VMEM((1, B), f32),
            pltpu.VMEM((1, B), f32), pltpu.VMEM((1, B), f32),
            pltpu.VMEM((K + 1, B), f32), pltpu.VMEM((K + 1, B), jnp.int32),
            pltpu.VMEM((1, F), f32), pltpu.VMEM((1, 1), jnp.int32),
        ],
    )
    mx, mean, sm, attn, topi = phase1(
        xp, bc, bnc,
        gs[0][0], gs[0][1], gb0.reshape(1, F),
        gs[1][0], gs[1][1], gb1.reshape(1, F),
        gs[2][0], gs[2][1], gb2.reshape(1, F),
        gs[3][0], gs[3][1], gb3.reshape(1, F),
        gWo.reshape(1, F), gbo.reshape(1, 1))

    fW0h, fW0l = _wsplit(fW0.reshape(7, F, F))
    f1 = _wsplit(fW1)
    f2 = _wsplit(fW2)
    f3 = _wsplit(fW3)
    fo = _wsplit(fWo)
    phase2 = pl.pallas_call(
        _phase2_body,
        grid=(1,),
        in_specs=[
            _const((NPAD, F)),
            pl.BlockSpec(memory_space=pltpu.SMEM),
            _const((B, F)), _const((B, F)), _const((B, F)), _const((B, F)),
            _const((7, F, F)), _const((7, F, F)), _const((1, F)),
            _const((F, F)), _const((F, F)), _const((1, F)),
            _const((F, F)), _const((F, F)), _const((1, F)),
            _const((F, F)), _const((F, F)), _const((1, F)),
            _const((F, F)), _const((F, F)), _const((1, F)),
        ],
        out_specs=_const((B, F)),
        out_shape=jax.ShapeDtypeStruct((B, F), f32),
        scratch_shapes=[
            pltpu.VMEM((B, F), f32), pltpu.VMEM((B, F), f32),
            pltpu.VMEM((B, F), f32),
        ],
    )
    out = phase2(
        xp, topi, mx, mean, sm, attn,
        fW0h, fW0l, fb0.reshape(1, F),
        f1[0], f1[1], fb1.reshape(1, F),
        f2[0], f2[1], fb2.reshape(1, F),
        f3[0], f3[1], fb3.reshape(1, F),
        fo[0], fo[1], fbo.reshape(1, F))
    return out
